# Initial kernel scaffold; baseline (speedup 1.0000x reference)
import jax
import jax.numpy as jnp
from jax.experimental import pallas as pl

N = 1024
RU = 64
B = 32
M = 3
BB = 8
GRID = B // BB


def _prep_body(adj_ref, sup_ref):
    a = adj_ref[...]
    at = a.T
    d = jnp.sum(at, axis=0, keepdims=True)
    sup_ref[...] = at / d


def _cell_body(sup_ref, xin_ref, h_ref, wgh_ref, wgi_ref, bg_ref,
               wch_ref, wci_ref, bc_ref, wp_ref, bp_ref,
               out_ref, hout_ref):
    S = sup_ref[...]
    xin0 = xin_ref[...]
    H0 = h_ref[...]

    def spmm(x):
        return jnp.dot(S, x, preferred_element_type=jnp.float32)

    xin1 = spmm(xin0)
    xin2 = 2.0 * spmm(xin1) - xin0
    xins = (xin0, xin1, xin2)

    def gconv(Hp, wh_ref, wi_ref, b_ref):
        H1 = spmm(Hp)
        H2 = 2.0 * spmm(H1) - Hp
        wi = wi_ref[...]
        acc = None
        for m, Hm in enumerate((Hp, H1, H2)):
            t = jnp.dot(Hm.reshape(N * BB, RU), wh_ref[m * RU:(m + 1) * RU],
                        preferred_element_type=jnp.float32)
            t = t + xins[m].reshape(N * BB, 1) * wi[m:m + 1]
            acc = t if acc is None else acc + t
        return acc + b_ref[...]

    value = jax.nn.sigmoid(gconv(H0, wgh_ref, wgi_ref, bg_ref))
    r = value[:, :RU]
    u = value[:, RU:]
    Hr = H0.reshape(N * BB, RU)
    rH = (r * Hr).reshape(N, BB * RU)
    c = jnp.tanh(gconv(rH, wch_ref, wci_ref, bc_ref))
    nh = u * Hr + (1.0 - u) * c
    hout_ref[...] = nh.reshape(N, BB * RU)

    proj = jnp.sum(nh * wp_ref[...], axis=1, keepdims=True) + bp_ref[...]
    out_ref[...] = proj.reshape(N, BB).T


def kernel(inputs, hidden_state, adj, W_gate, b_gate, W_cand, b_cand,
           W_proj, b_proj):
    sup = pl.pallas_call(
        _prep_body,
        out_shape=jax.ShapeDtypeStruct((N, N), jnp.float32),
    )(adj)

    xin_t = inputs.T
    h_t = hidden_state[0].reshape(B, N, RU).transpose(1, 0, 2).reshape(N, B * RU)

    wg = W_gate.reshape(RU + 1, M, 2 * RU)
    wgh = wg[1:].transpose(1, 0, 2).reshape(M * RU, 2 * RU)
    wgi = wg[0]
    wc = W_cand.reshape(RU + 1, M, RU)
    wch = wc[1:].transpose(1, 0, 2).reshape(M * RU, RU)
    wci = wc[0]

    out_bn, hout_t = pl.pallas_call(
        _cell_body,
        grid=(GRID,),
        in_specs=[
            pl.BlockSpec((N, N), lambda i: (0, 0)),
            pl.BlockSpec((N, BB), lambda i: (0, i)),
            pl.BlockSpec((N, BB * RU), lambda i: (0, i)),
            pl.BlockSpec((M * RU, 2 * RU), lambda i: (0, 0)),
            pl.BlockSpec((M, 2 * RU), lambda i: (0, 0)),
            pl.BlockSpec((1, 2 * RU), lambda i: (0, 0)),
            pl.BlockSpec((M * RU, RU), lambda i: (0, 0)),
            pl.BlockSpec((M, RU), lambda i: (0, 0)),
            pl.BlockSpec((1, RU), lambda i: (0, 0)),
            pl.BlockSpec((1, RU), lambda i: (0, 0)),
            pl.BlockSpec((1, 1), lambda i: (0, 0)),
        ],
        out_specs=[
            pl.BlockSpec((BB, N), lambda i: (i, 0)),
            pl.BlockSpec((N, BB * RU), lambda i: (0, i)),
        ],
        out_shape=[
            jax.ShapeDtypeStruct((B, N), jnp.float32),
            jax.ShapeDtypeStruct((N, B * RU), jnp.float32),
        ],
    )(sup, xin_t, h_t, wgh, wgi, b_gate.reshape(1, 2 * RU),
      wch, wci, b_cand.reshape(1, RU), W_proj.reshape(1, RU),
      b_proj.reshape(1, 1))

    nh_b = hout_t.reshape(N, B, RU).transpose(1, 0, 2).reshape(B, N * RU)
    return out_bn, nh_b[None]


# speedup vs baseline: 3.0606x; 3.0606x over previous
import jax
import jax.numpy as jnp
from jax.experimental import pallas as pl

N = 1024
RU = 64
B = 32
M = 3
BB = 4
GRID = B // BB


def _prep_body(adj_ref, sup_ref):
    a = adj_ref[...]
    at = a.T
    d = jnp.sum(at, axis=0, keepdims=True)
    sup_ref[...] = at / d


def _cell_body(sup_ref, xin_ref, h_ref, wg_ref, bg_ref, wc_ref, bc_ref,
               wp_ref, bp_ref, out_ref, hout_ref):
    S = sup_ref[...]
    xin0 = xin_ref[0]
    H0 = h_ref[...]

    def spmm(x):
        return jnp.dot(S, x, preferred_element_type=jnp.float32)

    xin1 = spmm(xin0)
    xin2 = 2.0 * spmm(xin1) - xin0

    H1 = spmm(H0)
    H2 = 2.0 * spmm(H1) - H0
    Xg = jnp.concatenate([H0, H1, H2, xin0, xin1, xin2], axis=1)
    value = jax.nn.sigmoid(
        jnp.dot(Xg, wg_ref[...], preferred_element_type=jnp.float32)
        + bg_ref[...])
    r = value[:, :BB * RU]
    u = value[:, BB * RU:]

    rH = r * H0
    R1 = spmm(rH)
    R2 = 2.0 * spmm(R1) - rH
    Xc = jnp.concatenate([rH, R1, R2, xin0, xin1, xin2], axis=1)
    c = jnp.tanh(
        jnp.dot(Xc, wc_ref[...], preferred_element_type=jnp.float32)
        + bc_ref[...])

    nh = u * H0 + (1.0 - u) * c
    hout_ref[...] = nh
    pj = jnp.dot(nh, wp_ref[...], preferred_element_type=jnp.float32)
    out_ref[0] = pj.T + bp_ref[...]


def _block_weights(W, b, out_dim):
    eye = jnp.eye(BB, dtype=W.dtype)
    w = W.reshape(RU + 1, M, out_dim)
    wh = w[1:]
    wi = w[0]
    bh = jnp.einsum('bd,cmo->mbcdo', eye, wh).reshape(M * BB * RU, BB * out_dim)
    bi = jnp.einsum('bd,mo->mbdo', eye, wi).reshape(M * BB, BB * out_dim)
    Wbig = jnp.concatenate([bh, bi], axis=0)
    bbig = jnp.tile(b, (BB,)).reshape(1, BB * out_dim)
    return Wbig, bbig


def kernel(inputs, hidden_state, adj, W_gate, b_gate, W_cand, b_cand,
           W_proj, b_proj):
    sup = pl.pallas_call(
        _prep_body,
        out_shape=jax.ShapeDtypeStruct((N, N), jnp.float32),
    )(adj)

    xin_t = inputs.T.reshape(N, GRID, BB).transpose(1, 0, 2)
    h_t = hidden_state[0].reshape(B, N, RU).transpose(1, 0, 2).reshape(N, B * RU)

    wg_big, bg_big = _block_weights(W_gate, b_gate, 2 * RU)
    cols = (jnp.arange(BB)[:, None] * 2 * RU + jnp.arange(RU)[None, :]).reshape(-1)
    perm = jnp.concatenate([cols, cols + RU])
    wg_big = wg_big[:, perm]
    bg_big = bg_big[:, perm]
    wc_big, bc_big = _block_weights(W_cand, b_cand, RU)
    wp_big = jnp.kron(jnp.eye(BB, dtype=W_proj.dtype), W_proj)

    out_bn, hout_t = pl.pallas_call(
        _cell_body,
        grid=(GRID,),
        in_specs=[
            pl.BlockSpec((N, N), lambda i: (0, 0)),
            pl.BlockSpec((1, N, BB), lambda i: (i, 0, 0)),
            pl.BlockSpec((N, BB * RU), lambda i: (0, i)),
            pl.BlockSpec((M * BB * (RU + 1), BB * 2 * RU), lambda i: (0, 0)),
            pl.BlockSpec((1, BB * 2 * RU), lambda i: (0, 0)),
            pl.BlockSpec((M * BB * (RU + 1), BB * RU), lambda i: (0, 0)),
            pl.BlockSpec((1, BB * RU), lambda i: (0, 0)),
            pl.BlockSpec((BB * RU, BB), lambda i: (0, 0)),
            pl.BlockSpec((1, 1), lambda i: (0, 0)),
        ],
        out_specs=[
            pl.BlockSpec((1, BB, N), lambda i: (i, 0, 0)),
            pl.BlockSpec((N, BB * RU), lambda i: (0, i)),
        ],
        out_shape=[
            jax.ShapeDtypeStruct((GRID, BB, N), jnp.float32),
            jax.ShapeDtypeStruct((N, B * RU), jnp.float32),
        ],
    )(sup, xin_t, h_t, wg_big, bg_big, wc_big, bc_big,
      wp_big, b_proj.reshape(1, 1))

    nh_b = hout_t.reshape(N, B, RU).transpose(1, 0, 2).reshape(B, N * RU)
    return out_bn.reshape(B, N), nh_b[None]


# speedup vs baseline: 3.1068x; 1.0151x over previous
import jax
import jax.numpy as jnp
from jax.experimental import pallas as pl

N = 1024
RU = 64
B = 32
M = 3
BB = 4
GRID = B // BB
F32 = jnp.float32
BF16 = jnp.bfloat16


def _prep_body(adj_ref, xin_ref, sup_ref, x1_ref, x2_ref):
    a = adj_ref[...]
    at = a.T
    d = jnp.sum(at, axis=0, keepdims=True)
    sup = (at / d).astype(BF16)
    sup_ref[...] = sup
    x0 = xin_ref[...].astype(BF16)
    x1 = jnp.dot(sup, x0, preferred_element_type=F32)
    x2 = 2.0 * jnp.dot(sup, x1.astype(BF16), preferred_element_type=F32) \
        - x0.astype(F32)
    x1_ref[...] = x1.astype(BF16)
    x2_ref[...] = x2.astype(BF16)


def _cell_body(sup_ref, x0_ref, x1_ref, x2_ref, h_ref, wg_ref, bg_ref,
               wc_ref, bc_ref, wp_ref, bp_ref, out_ref, hout_ref):
    S = sup_ref[...]
    H0 = h_ref[...]
    H0b = H0.astype(BF16)

    def spmm(x):
        return jnp.dot(S, x, preferred_element_type=F32)

    H1 = spmm(H0b)
    H2 = 2.0 * spmm(H1.astype(BF16)) - H0
    Xg = jnp.concatenate(
        [H0b, H1.astype(BF16), H2.astype(BF16),
         x0_ref[0], x1_ref[0], x2_ref[0]], axis=1)
    value = jax.nn.sigmoid(
        jnp.dot(Xg, wg_ref[...], preferred_element_type=F32)
        + bg_ref[...])
    r = value[:, :BB * RU]
    u = value[:, BB * RU:]

    rH = r * H0
    rHb = rH.astype(BF16)
    R1 = spmm(rHb)
    R2 = 2.0 * spmm(R1.astype(BF16)) - rH
    Xc = jnp.concatenate(
        [rHb, R1.astype(BF16), R2.astype(BF16),
         x0_ref[0], x1_ref[0], x2_ref[0]], axis=1)
    c = jnp.tanh(
        jnp.dot(Xc, wc_ref[...], preferred_element_type=F32)
        + bc_ref[...])

    nh = u * H0 + (1.0 - u) * c
    hout_ref[...] = nh
    pj = jnp.dot(nh.astype(BF16), wp_ref[...], preferred_element_type=F32)
    out_ref[0] = pj.T + bp_ref[...]


def _block_weights(W, b, out_dim):
    eye = jnp.eye(BB, dtype=F32)
    w = W.reshape(RU + 1, M, out_dim)
    wh = w[1:]
    wi = w[0]
    bh = jnp.einsum('bd,cmo->mbcdo', eye, wh).reshape(M * BB * RU, BB * out_dim)
    bi = jnp.einsum('bd,mo->mbdo', eye, wi).reshape(M * BB, BB * out_dim)
    Wbig = jnp.concatenate([bh, bi], axis=0)
    bbig = jnp.tile(b, (BB,)).reshape(1, BB * out_dim)
    return Wbig, bbig


def kernel(inputs, hidden_state, adj, W_gate, b_gate, W_cand, b_cand,
           W_proj, b_proj):
    xin_t = inputs.T
    sup, x1_t, x2_t = pl.pallas_call(
        _prep_body,
        out_shape=[
            jax.ShapeDtypeStruct((N, N), BF16),
            jax.ShapeDtypeStruct((N, B), BF16),
            jax.ShapeDtypeStruct((N, B), BF16),
        ],
    )(adj, xin_t)

    def _split(x):
        return x.reshape(N, GRID, BB).transpose(1, 0, 2)

    x0_r = _split(xin_t.astype(BF16))
    x1_r = _split(x1_t)
    x2_r = _split(x2_t)
    h_t = hidden_state[0].reshape(B, N, RU).transpose(1, 0, 2).reshape(N, B * RU)

    wg_big, bg_big = _block_weights(W_gate, b_gate, 2 * RU)
    cols = (jnp.arange(BB)[:, None] * 2 * RU + jnp.arange(RU)[None, :]).reshape(-1)
    perm = jnp.concatenate([cols, cols + RU])
    wg_big = wg_big[:, perm].astype(BF16)
    bg_big = bg_big[:, perm]
    wc_big, bc_big = _block_weights(W_cand, b_cand, RU)
    wc_big = wc_big.astype(BF16)
    wp_big = jnp.kron(jnp.eye(BB, dtype=F32), W_proj).astype(BF16)

    xspec = pl.BlockSpec((1, N, BB), lambda i: (i, 0, 0))
    out_bn, hout_t = pl.pallas_call(
        _cell_body,
        grid=(GRID,),
        in_specs=[
            pl.BlockSpec((N, N), lambda i: (0, 0)),
            xspec, xspec, xspec,
            pl.BlockSpec((N, BB * RU), lambda i: (0, i)),
            pl.BlockSpec((M * BB * (RU + 1), BB * 2 * RU), lambda i: (0, 0)),
            pl.BlockSpec((1, BB * 2 * RU), lambda i: (0, 0)),
            pl.BlockSpec((M * BB * (RU + 1), BB * RU), lambda i: (0, 0)),
            pl.BlockSpec((1, BB * RU), lambda i: (0, 0)),
            pl.BlockSpec((BB * RU, BB), lambda i: (0, 0)),
            pl.BlockSpec((1, 1), lambda i: (0, 0)),
        ],
        out_specs=[
            pl.BlockSpec((1, BB, N), lambda i: (i, 0, 0)),
            pl.BlockSpec((N, BB * RU), lambda i: (0, i)),
        ],
        out_shape=[
            jax.ShapeDtypeStruct((GRID, BB, N), F32),
            jax.ShapeDtypeStruct((N, B * RU), F32),
        ],
    )(sup, x0_r, x1_r, x2_r, h_t, wg_big, bg_big, wc_big, bc_big,
      wp_big, b_proj.reshape(1, 1))

    nh_b = hout_t.reshape(N, B, RU).transpose(1, 0, 2).reshape(B, N * RU)
    return out_bn.reshape(B, N), nh_b[None]


# speedup vs baseline: 4.7386x; 1.5252x over previous
import jax
import jax.numpy as jnp
from jax.experimental import pallas as pl

N = 1024
RU = 64
B = 32
M = 3
BB = 4
GRID = B // BB
F32 = jnp.float32
BF16 = jnp.bfloat16


def _prep_body(adj_ref, xin_ref, sup_ref, x0_ref, x1_ref, x2_ref):
    a = adj_ref[...]
    at = a.T
    d = jnp.sum(at, axis=0, keepdims=True)
    sup = (at / d).astype(BF16)
    sup_ref[...] = sup
    x0f = xin_ref[...]
    x0 = x0f.astype(BF16)
    x1 = jnp.dot(sup, x0, preferred_element_type=F32)
    x1b = x1.astype(BF16)
    x2b = (2.0 * jnp.dot(sup, x1b, preferred_element_type=F32) - x0f).astype(BF16)
    for i in range(GRID):
        sl = slice(i * BB, (i + 1) * BB)
        x0_ref[i] = x0[:, sl]
        x1_ref[i] = x1b[:, sl]
        x2_ref[i] = x2b[:, sl]


def _cell_body(sup_ref, x0_ref, x1_ref, x2_ref, h_ref, wg_ref, bg_ref,
               wc_ref, bc_ref, wp_ref, bp_ref, out_ref, hout_ref):
    S = sup_ref[...]
    H0 = jnp.concatenate([h_ref[b] for b in range(BB)], axis=1)
    H0b = H0.astype(BF16)

    def spmm(x):
        return jnp.dot(S, x, preferred_element_type=F32)

    H1 = spmm(H0b)
    H2 = 2.0 * spmm(H1.astype(BF16)) - H0
    Xg = jnp.concatenate(
        [H0b, H1.astype(BF16), H2.astype(BF16),
         x0_ref[0], x1_ref[0], x2_ref[0]], axis=1)
    bg = bg_ref[...]
    bias_g = jnp.concatenate([bg[:, :RU]] * BB + [bg[:, RU:]] * BB, axis=1)
    value = jax.nn.sigmoid(
        jnp.dot(Xg, wg_ref[...], preferred_element_type=F32)
        + bias_g)
    r = value[:, :BB * RU]
    u = value[:, BB * RU:]

    rH = r * H0
    rHb = rH.astype(BF16)
    R1 = spmm(rHb)
    R2 = 2.0 * spmm(R1.astype(BF16)) - rH
    Xc = jnp.concatenate(
        [rHb, R1.astype(BF16), R2.astype(BF16),
         x0_ref[0], x1_ref[0], x2_ref[0]], axis=1)
    bias_c = jnp.concatenate([bc_ref[...]] * BB, axis=1)
    c = jnp.tanh(
        jnp.dot(Xc, wc_ref[...], preferred_element_type=F32)
        + bias_c)

    nh = u * H0 + (1.0 - u) * c
    for b in range(BB):
        hout_ref[b] = nh[:, b * RU:(b + 1) * RU]
    pj = jnp.dot(nh.astype(BF16), wp_ref[...], preferred_element_type=F32)
    out_ref[0] = pj.T + bp_ref[...]


def _blockdiag(w):
    m, k, o = w.shape
    eye = jnp.eye(BB, dtype=w.dtype)
    big = w[:, None, :, None, :] * eye[None, :, None, :, None]
    return big.reshape(m * BB * k, BB * o)


def kernel(inputs, hidden_state, adj, W_gate, b_gate, W_cand, b_cand,
           W_proj, b_proj):
    xin_t = inputs.T
    xshape = jax.ShapeDtypeStruct((GRID, N, BB), BF16)
    sup, x0_r, x1_r, x2_r = pl.pallas_call(
        _prep_body,
        out_shape=[jax.ShapeDtypeStruct((N, N), BF16), xshape, xshape, xshape],
    )(adj, xin_t)

    h3 = hidden_state[0].reshape(B, N, RU)

    wg3 = W_gate.reshape(RU + 1, M, 2 * RU)
    whg = wg3[1:].transpose(1, 0, 2)
    wig = wg3[0][:, None, :]
    wg_big = jnp.concatenate(
        [jnp.concatenate([_blockdiag(whg[:, :, :RU]),
                          _blockdiag(whg[:, :, RU:])], axis=1),
         jnp.concatenate([_blockdiag(wig[:, :, :RU]),
                          _blockdiag(wig[:, :, RU:])], axis=1)],
        axis=0).astype(BF16)
    wc3 = W_cand.reshape(RU + 1, M, RU)
    wc_big = jnp.concatenate(
        [_blockdiag(wc3[1:].transpose(1, 0, 2)),
         _blockdiag(wc3[0][:, None, :])], axis=0).astype(BF16)
    wp_big = jnp.kron(jnp.eye(BB, dtype=F32), W_proj).astype(BF16)

    xspec = pl.BlockSpec((1, N, BB), lambda i: (i, 0, 0))
    out_bn, hout3 = pl.pallas_call(
        _cell_body,
        grid=(GRID,),
        in_specs=[
            pl.BlockSpec((N, N), lambda i: (0, 0)),
            xspec, xspec, xspec,
            pl.BlockSpec((BB, N, RU), lambda i: (i, 0, 0)),
            pl.BlockSpec((M * BB * (RU + 1), BB * 2 * RU), lambda i: (0, 0)),
            pl.BlockSpec((1, 2 * RU), lambda i: (0, 0)),
            pl.BlockSpec((M * BB * (RU + 1), BB * RU), lambda i: (0, 0)),
            pl.BlockSpec((1, RU), lambda i: (0, 0)),
            pl.BlockSpec((BB * RU, BB), lambda i: (0, 0)),
            pl.BlockSpec((1, 1), lambda i: (0, 0)),
        ],
        out_specs=[
            pl.BlockSpec((1, BB, N), lambda i: (i, 0, 0)),
            pl.BlockSpec((BB, N, RU), lambda i: (i, 0, 0)),
        ],
        out_shape=[
            jax.ShapeDtypeStruct((GRID, BB, N), F32),
            jax.ShapeDtypeStruct((B, N, RU), F32),
        ],
    )(sup, x0_r, x1_r, x2_r, h3, wg_big, b_gate.reshape(1, 2 * RU),
      wc_big, b_cand.reshape(1, RU), wp_big, b_proj.reshape(1, 1))

    return out_bn.reshape(B, N), hout3.reshape(1, B, N * RU)


# speedup vs baseline: 5.1397x; 1.0847x over previous
import jax
import jax.numpy as jnp
from jax.experimental import pallas as pl
from jax.experimental.pallas import tpu as pltpu

N = 1024
RU = 64
B = 32
M = 3
BB = 4
GRID = B // BB
F32 = jnp.float32
BF16 = jnp.bfloat16
DN = (((0,), (0,)), ((), ()))


def _body(adj_ref, xin_ref, h_ref, whg_ref, wig_ref, bg_ref,
          whc_ref, wic_ref, bc_ref, wp_ref, bp_ref,
          out_ref, hout_ref,
          sup_s, x_s, wg_s, wc_s, wp_s, bg_s, bc_s):
    i = pl.program_id(0)

    @pl.when(i == 0)
    def _prep():
        a = adj_ref[...]
        d = jnp.sum(a, axis=1, keepdims=True)
        sup_s[...] = (a * (1.0 / d)).astype(BF16)
        S = sup_s[...]
        x0f = xin_ref[...]
        x0 = x0f.astype(BF16)
        x1 = jax.lax.dot_general(S, x0, DN, preferred_element_type=F32)
        x1b = x1.astype(BF16)
        x2b = (2.0 * jax.lax.dot_general(S, x1b, DN, preferred_element_type=F32)
               - x0f).astype(BF16)
        for j in range(GRID):
            sl = slice(j * BB, (j + 1) * BB)
            x_s[j, :, 0 * BB:1 * BB] = x0[:, sl]
            x_s[j, :, 1 * BB:2 * BB] = x1b[:, sl]
            x_s[j, :, 2 * BB:3 * BB] = x2b[:, sl]

        wg_s[...] = jnp.zeros((M * BB * (RU + 1), 2 * BB * RU), BF16)
        wc_s[...] = jnp.zeros((M * BB * (RU + 1), BB * RU), BF16)
        wp_s[...] = jnp.zeros((BB * RU, BB), BF16)
        for m in range(M):
            hsl = slice(m * RU, (m + 1) * RU)
            xrow = M * BB * RU + m * BB
            for b in range(BB):
                rows = slice(m * BB * RU + b * RU, m * BB * RU + (b + 1) * RU)
                csl = slice(b * RU, (b + 1) * RU)
                usl = slice(BB * RU + b * RU, BB * RU + (b + 1) * RU)
                wg_s[rows, csl] = whg_ref[hsl, :RU]
                wg_s[rows, usl] = whg_ref[hsl, RU:]
                wc_s[rows, csl] = whc_ref[hsl, :]
                wg_s[xrow + b:xrow + b + 1, csl] = wig_ref[m:m + 1, :RU]
                wg_s[xrow + b:xrow + b + 1, usl] = wig_ref[m:m + 1, RU:]
                wc_s[xrow + b:xrow + b + 1, csl] = wic_ref[m:m + 1, :]
        for b in range(BB):
            csl = slice(b * RU, (b + 1) * RU)
            usl = slice(BB * RU + b * RU, BB * RU + (b + 1) * RU)
            wp_s[csl, b:b + 1] = wp_ref[...]
            bg_s[0:1, csl] = bg_ref[:, :RU]
            bg_s[0:1, usl] = bg_ref[:, RU:]
            bc_s[0:1, csl] = bc_ref[...]

    S = sup_s[...]

    def spmm(x):
        return jax.lax.dot_general(S, x, DN, preferred_element_type=F32)

    H0 = jnp.concatenate([h_ref[b] for b in range(BB)], axis=1)
    H0b = H0.astype(BF16)
    H1 = spmm(H0b)
    H1b = H1.astype(BF16)
    H2 = 2.0 * spmm(H1b) - H0
    xt = x_s[i]
    Xg = jnp.concatenate([H0b, H1b, H2.astype(BF16), xt], axis=1)
    value = jax.nn.sigmoid(
        jnp.dot(Xg, wg_s[...], preferred_element_type=F32) + bg_s[...])
    r = value[:, :BB * RU]
    u = value[:, BB * RU:]

    rH = r * H0
    rHb = rH.astype(BF16)
    R1 = spmm(rHb)
    R1b = R1.astype(BF16)
    R2 = 2.0 * spmm(R1b) - rH
    Xc = jnp.concatenate([rHb, R1b, R2.astype(BF16), xt], axis=1)
    c = jnp.tanh(
        jnp.dot(Xc, wc_s[...], preferred_element_type=F32) + bc_s[...])

    nh = u * H0 + (1.0 - u) * c
    for b in range(BB):
        hout_ref[b] = nh[:, b * RU:(b + 1) * RU]
    pj = jnp.dot(nh.astype(BF16), wp_s[...], preferred_element_type=F32)
    out_ref[0] = pj.T + bp_ref[...]


def kernel(inputs, hidden_state, adj, W_gate, b_gate, W_cand, b_cand,
           W_proj, b_proj):
    xin_t = inputs.T
    h3 = hidden_state[0].reshape(B, N, RU)

    wg3 = W_gate.reshape(RU + 1, M, 2 * RU)
    whg = wg3[1:].transpose(1, 0, 2).reshape(M * RU, 2 * RU).astype(BF16)
    wig = wg3[0].astype(BF16)
    wc3 = W_cand.reshape(RU + 1, M, RU)
    whc = wc3[1:].transpose(1, 0, 2).reshape(M * RU, RU).astype(BF16)
    wic = wc3[0].astype(BF16)

    const = lambda i: (0, 0)
    out_bn, hout3 = pl.pallas_call(
        _body,
        grid=(GRID,),
        in_specs=[
            pl.BlockSpec((N, N), const),
            pl.BlockSpec((N, B), const),
            pl.BlockSpec((BB, N, RU), lambda i: (i, 0, 0)),
            pl.BlockSpec((M * RU, 2 * RU), const),
            pl.BlockSpec((M, 2 * RU), const),
            pl.BlockSpec((1, 2 * RU), const),
            pl.BlockSpec((M * RU, RU), const),
            pl.BlockSpec((M, RU), const),
            pl.BlockSpec((1, RU), const),
            pl.BlockSpec((RU, 1), const),
            pl.BlockSpec((1, 1), const),
        ],
        out_specs=[
            pl.BlockSpec((1, BB, N), lambda i: (i, 0, 0)),
            pl.BlockSpec((BB, N, RU), lambda i: (i, 0, 0)),
        ],
        out_shape=[
            jax.ShapeDtypeStruct((GRID, BB, N), F32),
            jax.ShapeDtypeStruct((B, N, RU), F32),
        ],
        scratch_shapes=[
            pltpu.VMEM((N, N), BF16),
            pltpu.VMEM((GRID, N, M * BB), BF16),
            pltpu.VMEM((M * BB * (RU + 1), 2 * BB * RU), BF16),
            pltpu.VMEM((M * BB * (RU + 1), BB * RU), BF16),
            pltpu.VMEM((BB * RU, BB), BF16),
            pltpu.VMEM((1, 2 * BB * RU), F32),
            pltpu.VMEM((1, BB * RU), F32),
        ],
    )(adj, xin_t, h3, whg, wig, b_gate.reshape(1, 2 * RU),
      whc, wic, b_cand.reshape(1, RU), W_proj.astype(BF16),
      b_proj.reshape(1, 1))

    return out_bn.reshape(B, N), hout3.reshape(1, B, N * RU)
